# baseline (speedup 1.0000x reference)
import functools

import jax
import jax.numpy as jnp
from jax import lax
from jax.experimental import pallas as pl
from jax.experimental.pallas import tpu as pltpu
from jax.experimental.pallas import tpu_sc as plsc

D_MODEL = 768
NUM_EXPERTS = 64
TOKEN_BLOCK = 512
N_STREAMS = 16


def _logits_body(*refs):
    w = refs[N_STREAMS][...]
    l_ref = refs[N_STREAMS + 1]
    T = TOKEN_BLOCK
    for s in range(N_STREAMS):
        x = refs[s][...]
        logits = jax.lax.dot_general(
            x, w, (((1,), (1,)), ((), ())),
            preferred_element_type=jnp.float32)
        l_ref[:, s * T:(s + 1) * T] = logits.T


def _tc_logits(xf, W):
    N = xf.shape[0]
    T = TOKEN_BLOCK
    ns = N_STREAMS
    grid = (N // (T * ns),)

    def x_spec(s):
        return pl.BlockSpec((T, D_MODEL), lambda i, s=s: (ns * i + s, 0))

    return pl.pallas_call(
        _logits_body,
        grid=grid,
        in_specs=[x_spec(s) for s in range(ns)]
        + [pl.BlockSpec((NUM_EXPERTS, D_MODEL), lambda i: (0, 0))],
        out_specs=[pl.BlockSpec((NUM_EXPERTS, ns * T), lambda i: (0, i))],
        out_shape=[jax.ShapeDtypeStruct((NUM_EXPERTS, N), jnp.float32)],
    )(*([xf] * ns), W)[0]


def _sc_top2(logits_t):
    E, N = logits_t.shape
    info = plsc.get_sparse_core_info()
    NC, NS, L = info.num_cores, info.num_subcores, info.num_lanes
    NW = NC * NS
    C = N // NW
    mesh = plsc.VectorSubcoreMesh(core_axis_name="c", subcore_axis_name="s")

    @functools.partial(
        pl.kernel, mesh=mesh,
        out_type=[
            jax.ShapeDtypeStruct((2, N), jnp.float32),
            jax.ShapeDtypeStruct((2, N), jnp.int32),
        ],
        scratch_types=[
            pltpu.VMEM((E, C), jnp.float32),
            pltpu.VMEM((2, C), jnp.float32),
            pltpu.VMEM((2, C), jnp.int32),
        ],
    )
    def k(l_hbm, g_hbm, i_hbm, lv, gv, iv):
        wid = lax.axis_index("s") * NC + lax.axis_index("c")
        base = wid * C
        pltpu.sync_copy(l_hbm.at[:, pl.ds(base, C)], lv)

        def gloop(g, carry):
            m1 = jnp.full((L,), -jnp.inf, jnp.float32)
            m2 = jnp.full((L,), -jnp.inf, jnp.float32)
            i1 = jnp.zeros((L,), jnp.int32)
            i2 = jnp.zeros((L,), jnp.int32)
            for e in range(E):
                le = lv[e, pl.ds(g * L, L)]
                ev = jnp.full((L,), e, jnp.int32)
                gt1 = le > m1
                gt2 = le > m2
                i2 = jnp.where(gt1, i1, jnp.where(gt2, ev, i2))
                m2 = jnp.where(gt1, m1, jnp.where(gt2, le, m2))
                i1 = jnp.where(gt1, ev, i1)
                m1 = jnp.where(gt1, le, m1)
            g1 = 1.0 / (1.0 + jnp.exp(m2 - m1))
            gv[0, pl.ds(g * L, L)] = g1
            gv[1, pl.ds(g * L, L)] = 1.0 - g1
            iv[0, pl.ds(g * L, L)] = i1
            iv[1, pl.ds(g * L, L)] = i2
            return carry

        lax.fori_loop(0, C // L, gloop, 0)
        pltpu.sync_copy(gv, g_hbm.at[:, pl.ds(base, C)])
        pltpu.sync_copy(iv, i_hbm.at[:, pl.ds(base, C)])

    return k(logits_t)


def kernel(x, W):
    B, S, D = x.shape
    N = B * S
    xf = x.reshape(N, D)
    logits_t = _tc_logits(xf, W)
    gates_t, indices_t = _sc_top2(logits_t)
    gates = gates_t.T.reshape(B, S, 2)
    indices = indices_t.T.reshape(B, S, 2)
    return gates, indices


# speedup vs baseline: 1.0518x; 1.0518x over previous
import jax
import jax.numpy as jnp
from jax.experimental import pallas as pl

D_MODEL = 768
NUM_EXPERTS = 64
TOKEN_BLOCK = 512
N_STREAMS = 8


def _router_body(*refs):
    w = refs[N_STREAMS][...]
    g_ref = refs[N_STREAMS + 1]
    i_ref = refs[N_STREAMS + 2]
    T = TOKEN_BLOCK
    for s in range(N_STREAMS):
        x = refs[s][...]
        logits = jax.lax.dot_general(
            x, w, (((1,), (1,)), ((), ())),
            preferred_element_type=jnp.float32)
        i1 = jnp.argmax(logits, axis=-1)
        m1 = jnp.max(logits, axis=-1)
        iota = jax.lax.broadcasted_iota(jnp.int32, logits.shape, 1)
        masked = jnp.where(iota == i1[:, None], -jnp.inf, logits)
        i2 = jnp.argmax(masked, axis=-1)
        m2 = jnp.max(masked, axis=-1)
        g1 = 1.0 / (1.0 + jnp.exp(m2 - m1))
        g_ref[s * T:(s + 1) * T, :] = jnp.stack([g1, 1.0 - g1], axis=1)
        i_ref[s * T:(s + 1) * T, :] = jnp.stack([i1, i2], axis=1).astype(jnp.int32)


def kernel(x, W):
    B, S, D = x.shape
    N = B * S
    xf = x.reshape(N, D)
    T = TOKEN_BLOCK
    ns = N_STREAMS
    grid = (N // (T * ns),)

    def x_spec(s):
        return pl.BlockSpec((T, D), lambda i, s=s: (ns * i + s, 0))

    gates_t, indices_t = pl.pallas_call(
        _router_body,
        grid=grid,
        in_specs=[x_spec(s) for s in range(ns)]
        + [pl.BlockSpec((NUM_EXPERTS, D), lambda i: (0, 0))],
        out_specs=[
            pl.BlockSpec((ns * T, 2), lambda i: (i, 0)),
            pl.BlockSpec((ns * T, 2), lambda i: (i, 0)),
        ],
        out_shape=[
            jax.ShapeDtypeStruct((N, 2), jnp.float32),
            jax.ShapeDtypeStruct((N, 2), jnp.int32),
        ],
    )(*([xf] * ns), W)
    return gates_t.reshape(B, S, 2), indices_t.reshape(B, S, 2)


# speedup vs baseline: 1.7396x; 1.6539x over previous
import jax
import jax.numpy as jnp
from jax.experimental import pallas as pl

D_MODEL = 768
NUM_EXPERTS = 64
TOKEN_BLOCK = 256
N_STREAMS = 32


def _router_body(*refs):
    w = refs[N_STREAMS][...]
    g_ref = refs[N_STREAMS + 1]
    i_ref = refs[N_STREAMS + 2]
    T = TOKEN_BLOCK
    for s in range(N_STREAMS):
        x = refs[s][...]
        logits = jax.lax.dot_general(
            x, w, (((1,), (1,)), ((), ())),
            preferred_element_type=jnp.float32)
        i1 = jnp.argmax(logits, axis=-1)
        m1 = jnp.max(logits, axis=-1)
        iota = jax.lax.broadcasted_iota(jnp.int32, logits.shape, 1)
        masked = jnp.where(iota == i1[:, None], -jnp.inf, logits)
        i2 = jnp.argmax(masked, axis=-1)
        m2 = jnp.max(masked, axis=-1)
        g1 = 1.0 / (1.0 + jnp.exp(m2 - m1))
        g_ref[:, s * T:(s + 1) * T] = jnp.stack([g1, 1.0 - g1], axis=0)
        i_ref[:, s * T:(s + 1) * T] = jnp.stack([i1, i2], axis=0).astype(jnp.int32)


def kernel(x, W):
    B, S, D = x.shape
    N = B * S
    xf = x.reshape(N, D)
    T = TOKEN_BLOCK
    ns = N_STREAMS
    grid = (N // (T * ns),)

    def x_spec(s):
        return pl.BlockSpec((T, D), lambda i, s=s: (ns * i + s, 0))

    gates_t, indices_t = pl.pallas_call(
        _router_body,
        grid=grid,
        in_specs=[x_spec(s) for s in range(ns)]
        + [pl.BlockSpec((NUM_EXPERTS, D), lambda i: (0, 0))],
        out_specs=[
            pl.BlockSpec((2, ns * T), lambda i: (0, i)),
            pl.BlockSpec((2, ns * T), lambda i: (0, i)),
        ],
        out_shape=[
            jax.ShapeDtypeStruct((2, N), jnp.float32),
            jax.ShapeDtypeStruct((2, N), jnp.int32),
        ],
    )(*([xf] * ns), W)
    gates = gates_t.T.reshape(B, S, 2)
    indices = indices_t.T.reshape(B, S, 2)
    return gates, indices


# speedup vs baseline: 1.7653x; 1.0147x over previous
import jax
import jax.numpy as jnp
from jax.experimental import pallas as pl

D_MODEL = 768
NUM_EXPERTS = 64
TOKEN_BLOCK = 512
N_STREAMS = 16


def _router_body(*refs):
    w = refs[N_STREAMS][...]
    g_ref = refs[N_STREAMS + 1]
    i_ref = refs[N_STREAMS + 2]
    T = TOKEN_BLOCK
    for s in range(N_STREAMS):
        x = refs[s][...]
        logits = jax.lax.dot_general(
            x, w, (((1,), (1,)), ((), ())),
            preferred_element_type=jnp.float32)
        i1 = jnp.argmax(logits, axis=-1)
        m1 = jnp.max(logits, axis=-1)
        iota = jax.lax.broadcasted_iota(jnp.int32, logits.shape, 1)
        masked = jnp.where(iota == i1[:, None], -jnp.inf, logits)
        i2 = jnp.argmax(masked, axis=-1)
        m2 = jnp.max(masked, axis=-1)
        g1 = 1.0 / (1.0 + jnp.exp(m2 - m1))
        g_ref[:, s * T:(s + 1) * T] = jnp.stack([g1, 1.0 - g1], axis=0)
        i_ref[:, s * T:(s + 1) * T] = jnp.stack([i1, i2], axis=0).astype(jnp.int32)


def kernel(x, W):
    B, S, D = x.shape
    N = B * S
    xf = x.reshape(N, D)
    T = TOKEN_BLOCK
    ns = N_STREAMS
    grid = (N // (T * ns),)

    def x_spec(s):
        return pl.BlockSpec((T, D), lambda i, s=s: (ns * i + s, 0))

    gates_t, indices_t = pl.pallas_call(
        _router_body,
        grid=grid,
        in_specs=[x_spec(s) for s in range(ns)]
        + [pl.BlockSpec((NUM_EXPERTS, D), lambda i: (0, 0))],
        out_specs=[
            pl.BlockSpec((2, ns * T), lambda i: (0, i)),
            pl.BlockSpec((2, ns * T), lambda i: (0, i)),
        ],
        out_shape=[
            jax.ShapeDtypeStruct((2, N), jnp.float32),
            jax.ShapeDtypeStruct((2, N), jnp.int32),
        ],
    )(*([xf] * ns), W)
    gates = gates_t.T.reshape(B, S, 2)
    indices = indices_t.T.reshape(B, S, 2)
    return gates, indices
